# baseline (speedup 1.0000x reference)
import functools

import jax
import jax.numpy as jnp
from jax.experimental import pallas as pl
from jax.experimental.pallas import tpu as pltpu

N = 50000
E = 800000
DIN = 7
DE = 2
H = 64
L = 3
NH = 4
HD = H // NH

BN = 2000
BE = 4000


def _full(shape):
    return pl.BlockSpec(shape, lambda i: tuple(0 for _ in shape))


def _rows(bm, w):
    return pl.BlockSpec((bm, w), lambda i: (i, 0))


def _encode_nodes_body(nf_ref, w_ref, b_ref, deg_ref, freq_ref, out_ref):
    x = jnp.dot(nf_ref[...], w_ref[...], preferred_element_type=jnp.float32)
    x = jax.nn.relu(x + b_ref[...])
    ang = deg_ref[...] * freq_ref[...]
    pos = jnp.concatenate([jnp.sin(ang), jnp.cos(ang)], axis=1)
    out_ref[...] = x + pos


def _encode_nodes(nfp, wp, b, deg, freq):
    return pl.pallas_call(
        _encode_nodes_body,
        grid=(N // BN,),
        in_specs=[_rows(BN, 8), _full((8, H)), _full((1, H)),
                  _rows(BN, 1), _full((1, H // 2))],
        out_specs=_rows(BN, H),
        out_shape=jax.ShapeDtypeStruct((N, H), jnp.float32),
    )(nfp, wp, b, deg, freq)


def _encode_edges_body(ea_ref, w_ref, b_ref, ae_w_ref, e_ref, ae_ref):
    e = jnp.dot(ea_ref[...], w_ref[...], preferred_element_type=jnp.float32)
    e = jax.nn.relu(e + b_ref[...])
    e_ref[...] = e
    ae_ref[...] = jnp.dot(e, ae_w_ref[...], preferred_element_type=jnp.float32)


def _encode_edges(eap, wp, b, ae_w):
    return pl.pallas_call(
        _encode_edges_body,
        grid=(E // BE,),
        in_specs=[_rows(BE, 8), _full((8, H)), _full((1, H)), _full((H, 8))],
        out_specs=[_rows(BE, H), _rows(BE, 8)],
        out_shape=[jax.ShapeDtypeStruct((E, H), jnp.float32),
                   jax.ShapeDtypeStruct((E, 8), jnp.float32)],
    )(eap, wp, b, ae_w)


def _node_tables_body(x_ref, wv_ref, asd_ref, v_ref, t_ref):
    v = jnp.dot(x_ref[...], wv_ref[...], preferred_element_type=jnp.float32)
    v_ref[...] = v
    t_ref[...] = jnp.dot(v, asd_ref[...], preferred_element_type=jnp.float32)


def _node_tables(x, wv, a_sd):
    return pl.pallas_call(
        _node_tables_body,
        grid=(N // BN,),
        in_specs=[_rows(BN, H), _full((H, H)), _full((H, 8))],
        out_specs=[_rows(BN, H), _rows(BN, 8)],
        out_shape=[jax.ShapeDtypeStruct((N, H), jnp.float32),
                   jax.ShapeDtypeStruct((N, 8), jnp.float32)],
    )(x, wv, a_sd)


def _x_update_body(agg_ref, w_ref, b_ref, x_ref, out_ref):
    y = jnp.dot(agg_ref[...], w_ref[...], preferred_element_type=jnp.float32)
    out_ref[...] = jax.nn.relu(y + b_ref[...]) + x_ref[...]


def _x_update(agg, w, b, x):
    return pl.pallas_call(
        _x_update_body,
        grid=(N // BN,),
        in_specs=[_rows(BN, H), _full((H, H)), _full((1, H)), _rows(BN, H)],
        out_specs=_rows(BN, H),
        out_shape=jax.ShapeDtypeStruct((N, H), jnp.float32),
    )(agg, w, b, x)


def _e_update_body(g_ref, w_ref, ae_w_ref, e_ref, ae_ref):
    e = jax.nn.relu(jnp.dot(g_ref[...], w_ref[...],
                            preferred_element_type=jnp.float32))
    e_ref[...] = e
    ae_ref[...] = jnp.dot(e, ae_w_ref[...], preferred_element_type=jnp.float32)


def _e_update(g, w, ae_w):
    return pl.pallas_call(
        _e_update_body,
        grid=(E // BE,),
        in_specs=[_rows(BE, H), _full((H, H)), _full((H, 8))],
        out_specs=[_rows(BE, H), _rows(BE, 8)],
        out_shape=[jax.ShapeDtypeStruct((E, H), jnp.float32),
                   jax.ShapeDtypeStruct((E, 8), jnp.float32)],
    )(g, w, ae_w)


def _scorer_body(xs_ref, xd_ref, e_ref, w1s_ref, w1d_ref, w1e_ref,
                 b1_ref, w2_ref, b2_ref, out_ref):
    h = (jnp.dot(xs_ref[...], w1s_ref[...], preferred_element_type=jnp.float32)
         + jnp.dot(xd_ref[...], w1d_ref[...], preferred_element_type=jnp.float32)
         + jnp.dot(e_ref[...], w1e_ref[...], preferred_element_type=jnp.float32))
    h = jax.nn.relu(h + b1_ref[...])
    out_ref[...] = (jnp.dot(h, w2_ref[...], preferred_element_type=jnp.float32)
                    + b2_ref[...])


def _scorer(xs, xd, e, w1s, w1d, w1e, b1, w2, b2):
    return pl.pallas_call(
        _scorer_body,
        grid=(E // BE,),
        in_specs=[_rows(BE, H), _rows(BE, H), _rows(BE, H),
                  _full((H, H)), _full((H, H)), _full((H, H)),
                  _full((1, H)), _full((H, 8)), _full((1, 8))],
        out_specs=_rows(BE, 8),
        out_shape=jax.ShapeDtypeStruct((E, 8), jnp.float32),
    )(xs, xd, e, w1s, w1d, w1e, b1, w2, b2)


def kernel(node_features, edge_index, edge_attr, params):
    p = params
    src = edge_index[0]
    dst = edge_index[1]

    nfp = jnp.pad(node_features, ((0, 0), (0, 1)))
    node_wp = jnp.pad(p['node_W'], ((0, 1), (0, 0)))
    eap = jnp.pad(edge_attr, ((0, 0), (0, 6)))
    edge_wp = jnp.pad(p['edge_W'], ((0, 6), (0, 0)))

    def _blockdiag(att_l):
        cols = []
        for hh in range(NH):
            col = jnp.zeros((H,), jnp.float32).at[hh * HD:(hh + 1) * HD].set(att_l[hh])
            cols.append(col)
        m = jnp.stack(cols, axis=1)
        return jnp.pad(m, ((0, 0), (0, 8 - NH)))

    deg = jax.ops.segment_sum(jnp.ones((E,), jnp.float32), dst, num_segments=N)
    i = jnp.arange(H // 2, dtype=jnp.float32)
    freq = jnp.exp(-jnp.log(10000.0) * (2.0 * i / H))[None, :]

    x = _encode_nodes(nfp, node_wp, p['node_b'][None, :], deg[:, None], freq)
    ae_w0 = _blockdiag(p['att_edge'][0])
    e, ae = _encode_edges(eap, edge_wp, p['edge_b'][None, :], ae_w0)

    for l in range(L):
        a_sd = jnp.concatenate(
            [_blockdiag(p['att_src'][l])[:, :NH],
             _blockdiag(p['att_dst'][l])[:, :NH]], axis=1)
        v, asad = _node_tables(x, p['W_v'][l], a_sd)

        score = jax.nn.leaky_relu(
            asad[src, :NH] + asad[dst, NH:] + ae[:, :NH], 0.2)
        smax = jax.ops.segment_max(score, dst, num_segments=N)
        smax = jnp.where(jnp.isfinite(smax), smax, 0.0)
        ex = jnp.exp(score - smax[dst])
        den = jax.ops.segment_sum(ex, dst, num_segments=N)
        alpha = ex / (den[dst] + 1e-16)

        m = v[src].reshape(E, NH, HD) + e.reshape(E, NH, HD)
        agg = jax.ops.segment_sum(
            (alpha[..., None] * m).reshape(E, H), dst, num_segments=N)

        x = _x_update(agg, p['W_out'][l], p['b_out'][l][None, :], x)

        g = e + x[src] + x[dst]
        if l + 1 < L:
            ae_w = _blockdiag(p['att_edge'][l + 1])
        e, ae = _e_update(g, p['W_eu'][l], ae_w if l + 1 < L else ae_w0)

    xs = x[src]
    xd = x[dst]
    logits8 = _scorer(xs, xd, e,
                      p['sW1'][:H], p['sW1'][H:2 * H], p['sW1'][2 * H:],
                      p['sb1'][None, :],
                      jnp.pad(p['sW2'], ((0, 0), (0, 7))),
                      jnp.pad(p['sb2'], (0, 7))[None, :])
    edge_logits = logits8[:, 0]
    return (x, edge_logits)


# speedup vs baseline: 15.6439x; 15.6439x over previous
import functools

import jax
import jax.numpy as jnp
from jax import lax
from jax.experimental import pallas as pl
from jax.experimental.pallas import tpu as pltpu
from jax.experimental.pallas import tpu_sc as plsc

_NC = 2
_NS = 16
_NW = _NC * _NS

N = 50000
E = 800000
DIN = 7
DE = 2
H = 64
L = 3
NH = 4
HD = H // NH

BN = 2000
BE = 4000
GC = 200


@functools.cache
def _sc_gather_fn(n_rows, d, r, c):
    b_per_w = r // _NW
    nchunks = b_per_w // c
    mesh = plsc.VectorSubcoreMesh(core_axis_name="c", subcore_axis_name="s")

    @functools.partial(
        pl.kernel, mesh=mesh,
        out_type=jax.ShapeDtypeStruct((r, d), jnp.float32),
        scratch_types=[
            pltpu.VMEM((b_per_w,), jnp.int32),
            pltpu.VMEM((c, d), jnp.float32), pltpu.VMEM((c, d), jnp.float32),
            pltpu.SemaphoreType.DMA, pltpu.SemaphoreType.DMA,
        ],
    )
    def gather(table_hbm, idx_hbm, out_hbm, idx_all, r0, r1, s0, s1):
        wid = lax.axis_index("s") * _NC + lax.axis_index("c")
        base = wid * b_per_w
        pltpu.sync_copy(idx_hbm.at[pl.ds(base, b_per_w)], idx_all)

        def body(j2, _):
            o0 = j2 * 2 * c
            o1 = o0 + c
            cp0 = pltpu.async_copy(
                table_hbm.at[idx_all.at[pl.ds(o0, c)]], r0, s0)
            cp1 = pltpu.async_copy(
                table_hbm.at[idx_all.at[pl.ds(o1, c)]], r1, s1)
            cp0.wait()
            pltpu.sync_copy(r0, out_hbm.at[pl.ds(base + o0, c)])
            cp1.wait()
            pltpu.sync_copy(r1, out_hbm.at[pl.ds(base + o1, c)])
            return _

        lax.fori_loop(0, nchunks // 2, body, 0)
        if nchunks % 2:
            o0 = (nchunks - 1) * c
            pltpu.async_copy(
                table_hbm.at[idx_all.at[pl.ds(o0, c)]], r0, s0).wait()
            pltpu.sync_copy(r0, out_hbm.at[pl.ds(base + o0, c)])

    return gather


def _sc_gather(table, idx):
    n_rows, d = table.shape
    (r,) = idx.shape
    return _sc_gather_fn(n_rows, d, r, GC)(table, idx)


def _full(shape):
    return pl.BlockSpec(shape, lambda i: tuple(0 for _ in shape))


def _rows(bm, w):
    return pl.BlockSpec((bm, w), lambda i: (i, 0))


def _g128(bm):
    return pl.BlockSpec((bm, 2 * H), lambda i: (i, 0))


def _encode_nodes_body(nf_ref, w_ref, b_ref, deg_ref, freq_ref, out_ref):
    x = jnp.dot(nf_ref[...], w_ref[...], preferred_element_type=jnp.float32)
    x = jax.nn.relu(x + b_ref[...])
    ang = deg_ref[...] * freq_ref[...]
    pos = jnp.concatenate([jnp.sin(ang), jnp.cos(ang)], axis=1)
    out_ref[...] = x + pos


def _encode_nodes(nfp, wp, b, deg, freq):
    return pl.pallas_call(
        _encode_nodes_body,
        grid=(N // BN,),
        in_specs=[_rows(BN, 8), _full((8, H)), _full((1, H)),
                  _rows(BN, 1), _full((1, H // 2))],
        out_specs=_rows(BN, H),
        out_shape=jax.ShapeDtypeStruct((N, H), jnp.float32),
    )(nfp, wp, b, deg, freq)


def _encode_edges_body(ea_ref, w_ref, b_ref, ae_w_ref, e_ref, ae_ref):
    e = jnp.dot(ea_ref[...], w_ref[...], preferred_element_type=jnp.float32)
    e = jax.nn.relu(e + b_ref[...])
    e_ref[...] = e
    ae_ref[...] = jnp.dot(e, ae_w_ref[...], preferred_element_type=jnp.float32,
                          precision=lax.Precision.HIGHEST)


def _encode_edges(eap, wp, b, ae_w):
    return pl.pallas_call(
        _encode_edges_body,
        grid=(E // BE,),
        in_specs=[_rows(BE, 8), _full((8, H)), _full((1, H)), _full((H, 8))],
        out_specs=[_rows(BE, H), _rows(BE, 8)],
        out_shape=[jax.ShapeDtypeStruct((E, H), jnp.float32),
                   jax.ShapeDtypeStruct((E, 8), jnp.float32)],
    )(eap, wp, b, ae_w)


def _make_tbl_body(x_ref, wb_ref, out_ref):
    x = x_ref[...]
    out_ref[...] = jnp.concatenate(
        [x, jnp.dot(x, wb_ref[...], preferred_element_type=jnp.float32)],
        axis=1)


def _make_tbl(x, wb):
    return pl.pallas_call(
        _make_tbl_body,
        grid=(N // BN,),
        in_specs=[_rows(BN, H), _full((H, H))],
        out_specs=_rows(BN, 2 * H),
        out_shape=jax.ShapeDtypeStruct((N, 2 * H), jnp.float32),
    )(x, wb)


def _score_body(vs_ref, vd_ref, ae_ref, as_ref, ad_ref, sc_ref, bm_ref):
    z = (jnp.dot(vs_ref[:, H:], as_ref[...], preferred_element_type=jnp.float32,
                 precision=lax.Precision.HIGHEST)
         + jnp.dot(vd_ref[:, H:], ad_ref[...], preferred_element_type=jnp.float32,
                   precision=lax.Precision.HIGHEST)
         + ae_ref[...])
    s = jnp.where(z >= 0, z, 0.2 * z)
    sc_ref[...] = s
    bm_ref[...] = jnp.max(s, axis=0)[None, None, :]


def _score(gsrc, gdst, ae8, a_s, a_d):
    return pl.pallas_call(
        _score_body,
        grid=(E // BE,),
        in_specs=[_g128(BE), _g128(BE), _rows(BE, 8),
                  _full((H, 8)), _full((H, 8))],
        out_specs=[_rows(BE, 8),
                   pl.BlockSpec((1, 1, 8), lambda i: (i, 0, 0))],
        out_shape=[jax.ShapeDtypeStruct((E, 8), jnp.float32),
                   jax.ShapeDtypeStruct((E // BE, 1, 8), jnp.float32)],
    )(gsrc, gdst, ae8, a_s, a_d)


def _msg_body(sc_ref, gmax_ref, vs_ref, e_ref, rep_ref, ex_ref, msg_ref):
    ex = jnp.exp(sc_ref[...] - gmax_ref[...])
    ex_ref[...] = ex[:, :NH]
    ex_rep = jnp.dot(ex, rep_ref[...], preferred_element_type=jnp.float32,
                     precision=lax.Precision.HIGHEST)
    msg_ref[...] = ex_rep * (vs_ref[:, H:] + e_ref[...])


def _msg(score8, gmax, gsrc, e, rep8):
    return pl.pallas_call(
        _msg_body,
        grid=(E // BE,),
        in_specs=[_rows(BE, 8), _full((1, 8)), _g128(BE), _rows(BE, H),
                  _full((8, H))],
        out_specs=[_rows(BE, NH), _rows(BE, H)],
        out_shape=[jax.ShapeDtypeStruct((E, NH), jnp.float32),
                   jax.ShapeDtypeStruct((E, H), jnp.float32)],
    )(score8, gmax, gsrc, e, rep8)


def _x_update_body(num_ref, den_ref, rep_ref, w_ref, b_ref, x_ref, out_ref):
    den_rep = jnp.dot(den_ref[...], rep_ref[...],
                      preferred_element_type=jnp.float32,
                      precision=lax.Precision.HIGHEST)
    agg = num_ref[...] / (den_rep + 1e-16)
    y = jnp.dot(agg, w_ref[...], preferred_element_type=jnp.float32)
    out_ref[...] = jax.nn.relu(y + b_ref[...]) + x_ref[...]


def _x_update(num, den, rep4, w, b, x):
    return pl.pallas_call(
        _x_update_body,
        grid=(N // BN,),
        in_specs=[_rows(BN, H), _rows(BN, NH), _full((NH, H)),
                  _full((H, H)), _full((1, H)), _rows(BN, H)],
        out_specs=_rows(BN, H),
        out_shape=jax.ShapeDtypeStruct((N, H), jnp.float32),
    )(num, den, rep4, w, b, x)


def _e_update_body(e_ref, xs_ref, xd_ref, w_ref, ae_w_ref, e2_ref, ae_ref):
    g = e_ref[...] + xs_ref[:, :H] + xd_ref[:, :H]
    e2 = jax.nn.relu(jnp.dot(g, w_ref[...],
                             preferred_element_type=jnp.float32))
    e2_ref[...] = e2
    ae_ref[...] = jnp.dot(e2, ae_w_ref[...],
                          preferred_element_type=jnp.float32,
                          precision=lax.Precision.HIGHEST)


def _e_update(e, gsrc, gdst, w, ae_w):
    return pl.pallas_call(
        _e_update_body,
        grid=(E // BE,),
        in_specs=[_rows(BE, H), _g128(BE), _g128(BE),
                  _full((H, H)), _full((H, 8))],
        out_specs=[_rows(BE, H), _rows(BE, 8)],
        out_shape=[jax.ShapeDtypeStruct((E, H), jnp.float32),
                   jax.ShapeDtypeStruct((E, 8), jnp.float32)],
    )(e, gsrc, gdst, w, ae_w)


def _scorer_body(us_ref, ud_ref, e_ref, w1e_ref, b1_ref, w2_ref, b2_ref,
                 out_ref):
    h = (us_ref[:, H:] + ud_ref[:, H:]
         + jnp.dot(e_ref[...], w1e_ref[...],
                   preferred_element_type=jnp.float32))
    h = jax.nn.relu(h + b1_ref[...])
    out_ref[...] = (jnp.dot(h, w2_ref[...], preferred_element_type=jnp.float32)
                    + b2_ref[...])


def _scorer(gsrc, gdst, e, w1e, b1, w2, b2):
    return pl.pallas_call(
        _scorer_body,
        grid=(E // BE,),
        in_specs=[_g128(BE), _g128(BE), _rows(BE, H),
                  _full((H, H)), _full((1, H)), _full((H, 8)), _full((1, 8))],
        out_specs=_rows(BE, 8),
        out_shape=jax.ShapeDtypeStruct((E, 8), jnp.float32),
    )(gsrc, gdst, e, w1e, b1, w2, b2)


def kernel(node_features, edge_index, edge_attr, params):
    p = params
    src = edge_index[0]
    dst = edge_index[1]

    nfp = jnp.pad(node_features, ((0, 0), (0, 1)))
    node_wp = jnp.pad(p['node_W'], ((0, 1), (0, 0)))
    eap = jnp.pad(edge_attr, ((0, 0), (0, 6)))
    edge_wp = jnp.pad(p['edge_W'], ((0, 6), (0, 0)))

    def _blockdiag(att_l):
        cols = [jnp.zeros((H,), jnp.float32).at[hh * HD:(hh + 1) * HD]
                .set(att_l[hh]) for hh in range(NH)]
        m = jnp.stack(cols, axis=1)
        return jnp.pad(m, ((0, 0), (0, 8 - NH)))

    rep8 = jnp.zeros((8, H), jnp.float32)
    for hh in range(NH):
        rep8 = rep8.at[hh, hh * HD:(hh + 1) * HD].set(1.0)
    rep4 = rep8[:NH]

    deg = jax.ops.segment_sum(jnp.ones((E,), jnp.float32), dst,
                              num_segments=N)
    i = jnp.arange(H // 2, dtype=jnp.float32)
    freq = jnp.exp(-jnp.log(10000.0) * (2.0 * i / H))[None, :]

    x = _encode_nodes(nfp, node_wp, p['node_b'][None, :], deg[:, None], freq)
    ae8 = None
    e, ae8 = _encode_edges(eap, edge_wp, p['edge_b'][None, :],
                           _blockdiag(p['att_edge'][0]))

    tbl = _make_tbl(x, p['W_v'][0])
    gsrc = _sc_gather(tbl, src)
    gdst = _sc_gather(tbl, dst)

    for l in range(L):
        score8, bm = _score(gsrc, gdst, ae8,
                            _blockdiag(p['att_src'][l]),
                            _blockdiag(p['att_dst'][l]))
        gmax = jnp.max(bm, axis=(0, 1))[None, :]
        ex4, msg = _msg(score8, gmax, gsrc, e, rep8)

        den = jax.ops.segment_sum(ex4, dst, num_segments=N)
        num = jax.ops.segment_sum(msg, dst, num_segments=N)
        x = _x_update(num, den, rep4, p['W_out'][l], p['b_out'][l][None, :],
                      x)

        if l + 1 < L:
            tbl = _make_tbl(x, p['W_v'][l + 1])
            gsrc2 = _sc_gather(tbl, src)
            gdst2 = _sc_gather(tbl, dst)
            ae_w_next = _blockdiag(p['att_edge'][l + 1])
        else:
            ts = _make_tbl(x, p['sW1'][:H])
            td = _make_tbl(x, p['sW1'][H:2 * H])
            gsrc2 = _sc_gather(ts, src)
            gdst2 = _sc_gather(td, dst)
            ae_w_next = _blockdiag(p['att_edge'][0])

        e, ae8 = _e_update(e, gsrc2, gdst2, p['W_eu'][l], ae_w_next)
        gsrc, gdst = gsrc2, gdst2

    logits8 = _scorer(gsrc, gdst, e, p['sW1'][2 * H:], p['sb1'][None, :],
                      jnp.pad(p['sW2'], ((0, 0), (0, 7))),
                      jnp.pad(p['sb2'], (0, 7))[None, :])
    edge_logits = logits8[:, 0]
    return (x, edge_logits)


# speedup vs baseline: 18.9113x; 1.2089x over previous
import functools

import jax
import jax.numpy as jnp
from jax import lax
from jax.experimental import pallas as pl
from jax.experimental.pallas import tpu as pltpu
from jax.experimental.pallas import tpu_sc as plsc

_NC = 2
_NS = 16
_NW = _NC * _NS

N = 50000
E = 800000
DIN = 7
DE = 2
H = 64
L = 3
NH = 4
HD = H // NH

BN = 2000
BE = 4000
GC = 200


@functools.cache
def _sc_gather_fn(n_rows, d, r, c):
    b_per_w = r // _NW
    nchunks = b_per_w // c
    mesh = plsc.VectorSubcoreMesh(core_axis_name="c", subcore_axis_name="s")

    @functools.partial(
        pl.kernel, mesh=mesh,
        out_type=jax.ShapeDtypeStruct((r, d), jnp.float32),
        scratch_types=[
            pltpu.VMEM((b_per_w,), jnp.int32),
            pltpu.VMEM((c, d), jnp.float32), pltpu.VMEM((c, d), jnp.float32),
            pltpu.SemaphoreType.DMA, pltpu.SemaphoreType.DMA,
        ],
    )
    def gather(table_hbm, idx_hbm, out_hbm, idx_all, r0, r1, s0, s1):
        wid = lax.axis_index("s") * _NC + lax.axis_index("c")
        base = wid * b_per_w
        pltpu.sync_copy(idx_hbm.at[pl.ds(base, b_per_w)], idx_all)

        def body(j2, _):
            o0 = j2 * 2 * c
            o1 = o0 + c
            cp0 = pltpu.async_copy(
                table_hbm.at[idx_all.at[pl.ds(o0, c)]], r0, s0)
            cp1 = pltpu.async_copy(
                table_hbm.at[idx_all.at[pl.ds(o1, c)]], r1, s1)
            cp0.wait()
            pltpu.sync_copy(r0, out_hbm.at[pl.ds(base + o0, c)])
            cp1.wait()
            pltpu.sync_copy(r1, out_hbm.at[pl.ds(base + o1, c)])
            return _

        lax.fori_loop(0, nchunks // 2, body, 0)
        if nchunks % 2:
            o0 = (nchunks - 1) * c
            pltpu.async_copy(
                table_hbm.at[idx_all.at[pl.ds(o0, c)]], r0, s0).wait()
            pltpu.sync_copy(r0, out_hbm.at[pl.ds(base + o0, c)])

    return gather


def _sc_gather(table, idx):
    n_rows, d = table.shape
    (r,) = idx.shape
    return _sc_gather_fn(n_rows, d, r, GC)(table, idx)


def _full(shape):
    return pl.BlockSpec(shape, lambda i: tuple(0 for _ in shape))


def _rows(bm, w):
    return pl.BlockSpec((bm, w), lambda i: (i, 0))


def _g128(bm):
    return pl.BlockSpec((bm, 2 * H), lambda i: (i, 0))


def _encode_nodes_body(nf_ref, w_ref, b_ref, deg_ref, freq_ref, out_ref):
    x = jnp.dot(nf_ref[...], w_ref[...], preferred_element_type=jnp.float32)
    x = jax.nn.relu(x + b_ref[...])
    ang = deg_ref[...] * freq_ref[...]
    pos = jnp.concatenate([jnp.sin(ang), jnp.cos(ang)], axis=1)
    out_ref[...] = x + pos


def _encode_nodes(nfp, wp, b, deg, freq):
    return pl.pallas_call(
        _encode_nodes_body,
        grid=(N // BN,),
        in_specs=[_rows(BN, 8), _full((8, H)), _full((1, H)),
                  _rows(BN, 1), _full((1, H // 2))],
        out_specs=_rows(BN, H),
        out_shape=jax.ShapeDtypeStruct((N, H), jnp.float32),
    )(nfp, wp, b, deg, freq)


def _encode_edges_body(ea_ref, w_ref, b_ref, ae_w_ref, e_ref, ae_ref):
    e = jnp.dot(ea_ref[...], w_ref[...], preferred_element_type=jnp.float32)
    e = jax.nn.relu(e + b_ref[...])
    e_ref[...] = e
    ae_ref[...] = jnp.dot(e, ae_w_ref[...], preferred_element_type=jnp.float32,
                          precision=lax.Precision.HIGHEST)


def _encode_edges(eap, wp, b, ae_w):
    return pl.pallas_call(
        _encode_edges_body,
        grid=(E // BE,),
        in_specs=[_rows(BE, 8), _full((8, H)), _full((1, H)), _full((H, 8))],
        out_specs=[_rows(BE, H), _rows(BE, 8)],
        out_shape=[jax.ShapeDtypeStruct((E, H), jnp.float32),
                   jax.ShapeDtypeStruct((E, 8), jnp.float32)],
    )(eap, wp, b, ae_w)


def _make_tbl_body(x_ref, wb_ref, out_ref):
    x = x_ref[...]
    out_ref[...] = jnp.concatenate(
        [x, jnp.dot(x, wb_ref[...], preferred_element_type=jnp.float32)],
        axis=1)


def _make_tbl(x, wb):
    return pl.pallas_call(
        _make_tbl_body,
        grid=(N // BN,),
        in_specs=[_rows(BN, H), _full((H, H))],
        out_specs=_rows(BN, 2 * H),
        out_shape=jax.ShapeDtypeStruct((N, 2 * H), jnp.float32),
    )(x, wb)


def _score_body(vs_ref, vd_ref, ae_ref, as_ref, ad_ref, sc_ref, bm_ref):
    z = (jnp.dot(vs_ref[:, H:], as_ref[...], preferred_element_type=jnp.float32,
                 precision=lax.Precision.HIGHEST)
         + jnp.dot(vd_ref[:, H:], ad_ref[...], preferred_element_type=jnp.float32,
                   precision=lax.Precision.HIGHEST)
         + ae_ref[...])
    s = jnp.where(z >= 0, z, 0.2 * z)
    sc_ref[...] = s
    bm_ref[...] = jnp.max(s, axis=0)[None, None, :]


def _score(gsrc, gdst, ae8, a_s, a_d):
    return pl.pallas_call(
        _score_body,
        grid=(E // BE,),
        in_specs=[_g128(BE), _g128(BE), _rows(BE, 8),
                  _full((H, 8)), _full((H, 8))],
        out_specs=[_rows(BE, 8),
                   pl.BlockSpec((1, 1, 8), lambda i: (i, 0, 0))],
        out_shape=[jax.ShapeDtypeStruct((E, 8), jnp.float32),
                   jax.ShapeDtypeStruct((E // BE, 1, 8), jnp.float32)],
    )(gsrc, gdst, ae8, a_s, a_d)


def _msg_body(sc_ref, gmax_ref, vs_ref, e_ref, rep_ref, out_ref):
    ex = jnp.exp(sc_ref[...] - gmax_ref[...])
    ex_rep = jnp.dot(ex, rep_ref[...], preferred_element_type=jnp.float32,
                     precision=lax.Precision.HIGHEST)
    msg = ex_rep * (vs_ref[:, H:] + e_ref[...])
    out_ref[...] = jnp.concatenate([msg, ex[:, :NH]], axis=1)


def _msg(score8, gmax, gsrc, e, rep8):
    return pl.pallas_call(
        _msg_body,
        grid=(E // BE,),
        in_specs=[_rows(BE, 8), _full((1, 8)), _g128(BE), _rows(BE, H),
                  _full((8, H))],
        out_specs=_rows(BE, H + NH),
        out_shape=jax.ShapeDtypeStruct((E, H + NH), jnp.float32),
    )(score8, gmax, gsrc, e, rep8)


def _x_update_body(ne_ref, rep_ref, w_ref, b_ref, x_ref, out_ref):
    den_rep = jnp.dot(ne_ref[:, H:], rep_ref[...],
                      preferred_element_type=jnp.float32,
                      precision=lax.Precision.HIGHEST)
    agg = ne_ref[:, :H] / (den_rep + 1e-16)
    y = jnp.dot(agg, w_ref[...], preferred_element_type=jnp.float32)
    out_ref[...] = jax.nn.relu(y + b_ref[...]) + x_ref[...]


def _x_update(num_ext, rep4, w, b, x):
    return pl.pallas_call(
        _x_update_body,
        grid=(N // BN,),
        in_specs=[_rows(BN, H + NH), _full((NH, H)),
                  _full((H, H)), _full((1, H)), _rows(BN, H)],
        out_specs=_rows(BN, H),
        out_shape=jax.ShapeDtypeStruct((N, H), jnp.float32),
    )(num_ext, rep4, w, b, x)


def _e_update_body(e_ref, xs_ref, xd_ref, w_ref, ae_w_ref, e2_ref, ae_ref):
    g = e_ref[...] + xs_ref[:, :H] + xd_ref[:, :H]
    e2 = jax.nn.relu(jnp.dot(g, w_ref[...],
                             preferred_element_type=jnp.float32))
    e2_ref[...] = e2
    ae_ref[...] = jnp.dot(e2, ae_w_ref[...],
                          preferred_element_type=jnp.float32,
                          precision=lax.Precision.HIGHEST)


def _e_update(e, gsrc, gdst, w, ae_w):
    return pl.pallas_call(
        _e_update_body,
        grid=(E // BE,),
        in_specs=[_rows(BE, H), _g128(BE), _g128(BE),
                  _full((H, H)), _full((H, 8))],
        out_specs=[_rows(BE, H), _rows(BE, 8)],
        out_shape=[jax.ShapeDtypeStruct((E, H), jnp.float32),
                   jax.ShapeDtypeStruct((E, 8), jnp.float32)],
    )(e, gsrc, gdst, w, ae_w)


def _scorer_body(us_ref, ud_ref, e_ref, w1e_ref, b1_ref, w2_ref, b2_ref,
                 out_ref):
    h = (us_ref[:, H:] + ud_ref[:, H:]
         + jnp.dot(e_ref[...], w1e_ref[...],
                   preferred_element_type=jnp.float32))
    h = jax.nn.relu(h + b1_ref[...])
    out_ref[...] = (jnp.dot(h, w2_ref[...], preferred_element_type=jnp.float32)
                    + b2_ref[...])


def _scorer(gsrc, gdst, e, w1e, b1, w2, b2):
    return pl.pallas_call(
        _scorer_body,
        grid=(E // BE,),
        in_specs=[_g128(BE), _g128(BE), _rows(BE, H),
                  _full((H, H)), _full((1, H)), _full((H, 8)), _full((1, 8))],
        out_specs=_rows(BE, 8),
        out_shape=jax.ShapeDtypeStruct((E, 8), jnp.float32),
    )(gsrc, gdst, e, w1e, b1, w2, b2)


def kernel(node_features, edge_index, edge_attr, params):
    p = params
    src = edge_index[0]
    dst = edge_index[1]

    nfp = jnp.pad(node_features, ((0, 0), (0, 1)))
    node_wp = jnp.pad(p['node_W'], ((0, 1), (0, 0)))
    eap = jnp.pad(edge_attr, ((0, 0), (0, 6)))
    edge_wp = jnp.pad(p['edge_W'], ((0, 6), (0, 0)))

    def _blockdiag(att_l):
        cols = [jnp.zeros((H,), jnp.float32).at[hh * HD:(hh + 1) * HD]
                .set(att_l[hh]) for hh in range(NH)]
        m = jnp.stack(cols, axis=1)
        return jnp.pad(m, ((0, 0), (0, 8 - NH)))

    rep8 = jnp.zeros((8, H), jnp.float32)
    for hh in range(NH):
        rep8 = rep8.at[hh, hh * HD:(hh + 1) * HD].set(1.0)
    rep4 = rep8[:NH]

    deg = jax.ops.segment_sum(jnp.ones((E,), jnp.float32), dst,
                              num_segments=N)
    i = jnp.arange(H // 2, dtype=jnp.float32)
    freq = jnp.exp(-jnp.log(10000.0) * (2.0 * i / H))[None, :]

    x = _encode_nodes(nfp, node_wp, p['node_b'][None, :], deg[:, None], freq)
    ae8 = None
    e, ae8 = _encode_edges(eap, edge_wp, p['edge_b'][None, :],
                           _blockdiag(p['att_edge'][0]))

    tbl = _make_tbl(x, p['W_v'][0])
    gsrc = _sc_gather(tbl, src)
    gdst = _sc_gather(tbl, dst)

    for l in range(L):
        score8, bm = _score(gsrc, gdst, ae8,
                            _blockdiag(p['att_src'][l]),
                            _blockdiag(p['att_dst'][l]))
        gmax = jnp.max(bm, axis=(0, 1))[None, :]
        msgex = _msg(score8, gmax, gsrc, e, rep8)

        num_ext = jax.ops.segment_sum(msgex, dst, num_segments=N)
        x = _x_update(num_ext, rep4, p['W_out'][l], p['b_out'][l][None, :],
                      x)

        if l + 1 < L:
            tbl = _make_tbl(x, p['W_v'][l + 1])
            gsrc2 = _sc_gather(tbl, src)
            gdst2 = _sc_gather(tbl, dst)
            ae_w_next = _blockdiag(p['att_edge'][l + 1])
        else:
            ts = _make_tbl(x, p['sW1'][:H])
            td = _make_tbl(x, p['sW1'][H:2 * H])
            gsrc2 = _sc_gather(ts, src)
            gdst2 = _sc_gather(td, dst)
            ae_w_next = _blockdiag(p['att_edge'][0])

        e, ae8 = _e_update(e, gsrc2, gdst2, p['W_eu'][l], ae_w_next)
        gsrc, gdst = gsrc2, gdst2

    logits8 = _scorer(gsrc, gdst, e, p['sW1'][2 * H:], p['sb1'][None, :],
                      jnp.pad(p['sW2'], ((0, 0), (0, 7))),
                      jnp.pad(p['sb2'], (0, 7))[None, :])
    edge_logits = logits8[:, 0]
    return (x, edge_logits)


# speedup vs baseline: 20.8050x; 1.1001x over previous
import functools

import jax
import jax.numpy as jnp
from jax import lax
from jax.experimental import pallas as pl
from jax.experimental.pallas import tpu as pltpu
from jax.experimental.pallas import tpu_sc as plsc

_NC = 2
_NS = 16
_NW = _NC * _NS

N = 50000
E = 800000
DIN = 7
DE = 2
H = 64
L = 3
NH = 4
HD = H // NH

BN = 2000
BE = 4000
GC = 200


@functools.cache
def _sc_gather_fn(n_rows, d, r, c):
    b_per_w = r // _NW
    nchunks = b_per_w // c
    mesh = plsc.VectorSubcoreMesh(core_axis_name="c", subcore_axis_name="s")

    @functools.partial(
        pl.kernel, mesh=mesh,
        out_type=jax.ShapeDtypeStruct((r, d), jnp.float32),
        scratch_types=[
            pltpu.VMEM((b_per_w,), jnp.int32),
            pltpu.VMEM((c, d), jnp.float32), pltpu.VMEM((c, d), jnp.float32),
            pltpu.SemaphoreType.DMA, pltpu.SemaphoreType.DMA,
        ],
    )
    def gather(table_hbm, idx_hbm, out_hbm, idx_all, r0, r1, s0, s1):
        wid = lax.axis_index("s") * _NC + lax.axis_index("c")
        base = wid * b_per_w
        pltpu.sync_copy(idx_hbm.at[pl.ds(base, b_per_w)], idx_all)

        def body(j2, _):
            o0 = j2 * 2 * c
            o1 = o0 + c
            cp0 = pltpu.async_copy(
                table_hbm.at[idx_all.at[pl.ds(o0, c)]], r0, s0)
            cp1 = pltpu.async_copy(
                table_hbm.at[idx_all.at[pl.ds(o1, c)]], r1, s1)
            cp0.wait()
            pltpu.sync_copy(r0, out_hbm.at[pl.ds(base + o0, c)])
            cp1.wait()
            pltpu.sync_copy(r1, out_hbm.at[pl.ds(base + o1, c)])
            return _

        lax.fori_loop(0, nchunks // 2, body, 0)
        if nchunks % 2:
            o0 = (nchunks - 1) * c
            pltpu.async_copy(
                table_hbm.at[idx_all.at[pl.ds(o0, c)]], r0, s0).wait()
            pltpu.sync_copy(r0, out_hbm.at[pl.ds(base + o0, c)])

    return gather


def _sc_gather(table, idx):
    n_rows, d = table.shape
    (r,) = idx.shape
    return _sc_gather_fn(n_rows, d, r, GC)(table, idx)


def _full(shape):
    return pl.BlockSpec(shape, lambda i: tuple(0 for _ in shape))


def _rows(bm, w):
    return pl.BlockSpec((bm, w), lambda i: (i, 0))


def _g128(bm):
    return pl.BlockSpec((bm, 2 * H), lambda i: (i, 0))


def _encode_nodes_body(nf_ref, w_ref, b_ref, deg_ref, freq_ref, out_ref):
    x = jnp.dot(nf_ref[...], w_ref[...], preferred_element_type=jnp.float32)
    x = jax.nn.relu(x + b_ref[...])
    ang = deg_ref[...] * freq_ref[...]
    pos = jnp.concatenate([jnp.sin(ang), jnp.cos(ang)], axis=1)
    out_ref[...] = x + pos


def _encode_nodes(nfp, wp, b, deg, freq):
    return pl.pallas_call(
        _encode_nodes_body,
        grid=(N // BN,),
        in_specs=[_rows(BN, 8), _full((8, H)), _full((1, H)),
                  _rows(BN, 1), _full((1, H // 2))],
        out_specs=_rows(BN, H),
        out_shape=jax.ShapeDtypeStruct((N, H), jnp.float32),
    )(nfp, wp, b, deg, freq)


def _ek0_body(ea_ref, w_ref, b_ref, aew_ref, gs_ref, gd_ref, as_ref, ad_ref,
              e_ref, sc_ref, mb_ref, bm_ref):
    e = jnp.dot(ea_ref[...], w_ref[...], preferred_element_type=jnp.float32)
    e = jax.nn.relu(e + b_ref[...])
    e_ref[...] = e
    ae = jnp.dot(e, aew_ref[...], preferred_element_type=jnp.float32,
                 precision=lax.Precision.HIGHEST)
    vs = gs_ref[:, H:]
    z = (jnp.dot(vs, as_ref[...], preferred_element_type=jnp.float32,
                 precision=lax.Precision.HIGHEST)
         + jnp.dot(gd_ref[:, H:], ad_ref[...],
                   preferred_element_type=jnp.float32,
                   precision=lax.Precision.HIGHEST)
         + ae)
    sc = jnp.where(z >= 0, z, 0.2 * z)
    sc_ref[...] = sc
    mb_ref[...] = vs + e
    bm_ref[...] = jnp.max(sc, axis=0)[None, None, :]


def _ek0(eap, wp, b, aew, gsrc, gdst, a_s, a_d):
    return pl.pallas_call(
        _ek0_body,
        grid=(E // BE,),
        in_specs=[_rows(BE, 8), _full((8, H)), _full((1, H)), _full((H, 8)),
                  _g128(BE), _g128(BE), _full((H, 8)), _full((H, 8))],
        out_specs=[_rows(BE, H), _rows(BE, 8), _rows(BE, H),
                   pl.BlockSpec((1, 1, 8), lambda i: (i, 0, 0))],
        out_shape=[jax.ShapeDtypeStruct((E, H), jnp.float32),
                   jax.ShapeDtypeStruct((E, 8), jnp.float32),
                   jax.ShapeDtypeStruct((E, H), jnp.float32),
                   jax.ShapeDtypeStruct((E // BE, 1, 8), jnp.float32)],
    )(eap, wp, b, aew, gsrc, gdst, a_s, a_d)


def _ek_body(e_ref, gs_ref, gd_ref, weu_ref, aew_ref, as_ref, ad_ref,
             e2_ref, sc_ref, mb_ref, bm_ref):
    g = e_ref[...] + gs_ref[:, :H] + gd_ref[:, :H]
    e2 = jax.nn.relu(jnp.dot(g, weu_ref[...],
                             preferred_element_type=jnp.float32))
    e2_ref[...] = e2
    ae = jnp.dot(e2, aew_ref[...], preferred_element_type=jnp.float32,
                 precision=lax.Precision.HIGHEST)
    vs = gs_ref[:, H:]
    z = (jnp.dot(vs, as_ref[...], preferred_element_type=jnp.float32,
                 precision=lax.Precision.HIGHEST)
         + jnp.dot(gd_ref[:, H:], ad_ref[...],
                   preferred_element_type=jnp.float32,
                   precision=lax.Precision.HIGHEST)
         + ae)
    sc = jnp.where(z >= 0, z, 0.2 * z)
    sc_ref[...] = sc
    mb_ref[...] = vs + e2
    bm_ref[...] = jnp.max(sc, axis=0)[None, None, :]


def _ek(e, gsrc, gdst, weu, aew, a_s, a_d):
    return pl.pallas_call(
        _ek_body,
        grid=(E // BE,),
        in_specs=[_rows(BE, H), _g128(BE), _g128(BE),
                  _full((H, H)), _full((H, 8)), _full((H, 8)), _full((H, 8))],
        out_specs=[_rows(BE, H), _rows(BE, 8), _rows(BE, H),
                   pl.BlockSpec((1, 1, 8), lambda i: (i, 0, 0))],
        out_shape=[jax.ShapeDtypeStruct((E, H), jnp.float32),
                   jax.ShapeDtypeStruct((E, 8), jnp.float32),
                   jax.ShapeDtypeStruct((E, H), jnp.float32),
                   jax.ShapeDtypeStruct((E // BE, 1, 8), jnp.float32)],
    )(e, gsrc, gdst, weu, aew, a_s, a_d)


def _make_tbl_body(x_ref, wb_ref, out_ref):
    x = x_ref[...]
    out_ref[...] = jnp.concatenate(
        [x, jnp.dot(x, wb_ref[...], preferred_element_type=jnp.float32)],
        axis=1)


def _make_tbl(x, wb):
    return pl.pallas_call(
        _make_tbl_body,
        grid=(N // BN,),
        in_specs=[_rows(BN, H), _full((H, H))],
        out_specs=_rows(BN, 2 * H),
        out_shape=jax.ShapeDtypeStruct((N, 2 * H), jnp.float32),
    )(x, wb)


def _msg_body(sc_ref, gmax_ref, mb_ref, rep_ref, out_ref):
    ex = jnp.exp(sc_ref[...] - gmax_ref[...])
    ex_rep = jnp.dot(ex, rep_ref[...], preferred_element_type=jnp.float32,
                     precision=lax.Precision.HIGHEST)
    out_ref[...] = jnp.concatenate([ex_rep * mb_ref[...], ex[:, :NH]], axis=1)


def _msg(score8, gmax, mbase, rep8):
    return pl.pallas_call(
        _msg_body,
        grid=(E // BE,),
        in_specs=[_rows(BE, 8), _full((1, 8)), _rows(BE, H), _full((8, H))],
        out_specs=_rows(BE, H + NH),
        out_shape=jax.ShapeDtypeStruct((E, H + NH), jnp.float32),
    )(score8, gmax, mbase, rep8)


def _x_update_body(ne_ref, rep_ref, w_ref, b_ref, x_ref, out_ref):
    den_rep = jnp.dot(ne_ref[:, H:], rep_ref[...],
                      preferred_element_type=jnp.float32,
                      precision=lax.Precision.HIGHEST)
    agg = ne_ref[:, :H] / (den_rep + 1e-16)
    y = jnp.dot(agg, w_ref[...], preferred_element_type=jnp.float32)
    out_ref[...] = jax.nn.relu(y + b_ref[...]) + x_ref[...]


def _x_update(num_ext, rep4, w, b, x):
    return pl.pallas_call(
        _x_update_body,
        grid=(N // BN,),
        in_specs=[_rows(BN, H + NH), _full((NH, H)),
                  _full((H, H)), _full((1, H)), _rows(BN, H)],
        out_specs=_rows(BN, H),
        out_shape=jax.ShapeDtypeStruct((N, H), jnp.float32),
    )(num_ext, rep4, w, b, x)


def _ek3_body(e_ref, gs_ref, gd_ref, weu_ref, w1e_ref, b1_ref, w2_ref,
              b2_ref, out_ref):
    g = e_ref[...] + gs_ref[:, :H] + gd_ref[:, :H]
    e2 = jax.nn.relu(jnp.dot(g, weu_ref[...],
                             preferred_element_type=jnp.float32))
    h = (gs_ref[:, H:] + gd_ref[:, H:]
         + jnp.dot(e2, w1e_ref[...], preferred_element_type=jnp.float32))
    h = jax.nn.relu(h + b1_ref[...])
    out_ref[...] = (jnp.dot(h, w2_ref[...], preferred_element_type=jnp.float32)
                    + b2_ref[...])


def _ek3(e, gsrc, gdst, weu, w1e, b1, w2, b2):
    return pl.pallas_call(
        _ek3_body,
        grid=(E // BE,),
        in_specs=[_rows(BE, H), _g128(BE), _g128(BE), _full((H, H)),
                  _full((H, H)), _full((1, H)), _full((H, 8)), _full((1, 8))],
        out_specs=_rows(BE, 8),
        out_shape=jax.ShapeDtypeStruct((E, 8), jnp.float32),
    )(e, gsrc, gdst, weu, w1e, b1, w2, b2)


def kernel(node_features, edge_index, edge_attr, params):
    p = params
    src = edge_index[0]
    dst = edge_index[1]

    nfp = jnp.pad(node_features, ((0, 0), (0, 1)))
    node_wp = jnp.pad(p['node_W'], ((0, 1), (0, 0)))
    eap = jnp.pad(edge_attr, ((0, 0), (0, 6)))
    edge_wp = jnp.pad(p['edge_W'], ((0, 6), (0, 0)))

    def _blockdiag(att_l):
        cols = [jnp.zeros((H,), jnp.float32).at[hh * HD:(hh + 1) * HD]
                .set(att_l[hh]) for hh in range(NH)]
        m = jnp.stack(cols, axis=1)
        return jnp.pad(m, ((0, 0), (0, 8 - NH)))

    rep8 = jnp.zeros((8, H), jnp.float32)
    for hh in range(NH):
        rep8 = rep8.at[hh, hh * HD:(hh + 1) * HD].set(1.0)
    rep4 = rep8[:NH]

    deg = jax.ops.segment_sum(jnp.ones((E,), jnp.float32), dst,
                              num_segments=N)
    i = jnp.arange(H // 2, dtype=jnp.float32)
    freq = jnp.exp(-jnp.log(10000.0) * (2.0 * i / H))[None, :]

    x = _encode_nodes(nfp, node_wp, p['node_b'][None, :], deg[:, None], freq)

    tbl = _make_tbl(x, p['W_v'][0])
    gsrc = _sc_gather(tbl, src)
    gdst = _sc_gather(tbl, dst)

    e = score8 = mbase = bm = None
    for l in range(L):
        if l == 0:
            e, score8, mbase, bm = _ek0(
                eap, edge_wp, p['edge_b'][None, :],
                _blockdiag(p['att_edge'][0]), gsrc, gdst,
                _blockdiag(p['att_src'][0]), _blockdiag(p['att_dst'][0]))
        gmax = jnp.max(bm, axis=(0, 1))[None, :]
        msgex = _msg(score8, gmax, mbase, rep8)

        num_ext = jax.ops.segment_sum(msgex, dst, num_segments=N)
        x = _x_update(num_ext, rep4, p['W_out'][l], p['b_out'][l][None, :],
                      x)

        if l + 1 < L:
            tbl = _make_tbl(x, p['W_v'][l + 1])
            gsrc = _sc_gather(tbl, src)
            gdst = _sc_gather(tbl, dst)
            e, score8, mbase, bm = _ek(
                e, gsrc, gdst, p['W_eu'][l],
                _blockdiag(p['att_edge'][l + 1]),
                _blockdiag(p['att_src'][l + 1]),
                _blockdiag(p['att_dst'][l + 1]))
        else:
            ts = _make_tbl(x, p['sW1'][:H])
            td = _make_tbl(x, p['sW1'][H:2 * H])
            gsrc = _sc_gather(ts, src)
            gdst = _sc_gather(td, dst)

    logits8 = _ek3(e, gsrc, gdst, p['W_eu'][L - 1], p['sW1'][2 * H:],
                   p['sb1'][None, :],
                   jnp.pad(p['sW2'], ((0, 0), (0, 7))),
                   jnp.pad(p['sb2'], (0, 7))[None, :])
    edge_logits = logits8[:, 0]
    return (x, edge_logits)
